# baseline (speedup 1.0000x reference)
import functools

import jax
import jax.numpy as jnp
from jax import lax
from jax.experimental import pallas as pl
from jax.experimental.pallas import tpu as pltpu
from jax.experimental.pallas import tpu_sc as plsc

N_FIELDS = 26
VOCAB = 100000
EMBED_DIM = 32
BATCH = 16384
HIDDEN = 512
OUT = 128
INPUT_DIM = N_FIELDS * EMBED_DIM

NC = 2
NS = 16
NW = NC * NS

PIECE = 4096
NPIECE = BATCH // PIECE

GROUPS = (7, 7, 6, 6)


def _sc_gather_t(features_t, tables_em, fg):
    mesh = plsc.VectorSubcoreMesh(core_axis_name="c", subcore_axis_name="s",
                                  num_cores=NC, num_subcores=NS)

    @functools.partial(
        pl.kernel,
        out_type=jax.ShapeDtypeStruct((fg * EMBED_DIM, BATCH), jnp.float32),
        mesh=mesh,
        scratch_types=[
            pltpu.VMEM((VOCAB,), jnp.float32),
            pltpu.VMEM((BATCH,), jnp.int32),
            [pltpu.VMEM((PIECE,), jnp.float32) for _ in range(2)],
            pltpu.SemaphoreType.DMA,
        ],
        compiler_params=pltpu.CompilerParams(use_tc_tiling_on_sc=False,
                                             needs_layout_passes=False),
    )
    def k(feat_hbm, tab_hbm, out_hbm, vvec, feat_v, piece_v, wsem):
        e = lax.axis_index("s") * NC + lax.axis_index("c")

        for f in range(fg):
            pltpu.sync_copy(tab_hbm.at[f, e], vvec)
            pltpu.sync_copy(feat_hbm.at[f], feat_v)
            row = f * EMBED_DIM + e
            wd = [None, None]
            for p in range(NPIECE):
                buf = piece_v[p % 2]
                if wd[p % 2] is not None:
                    wd[p % 2].wait()

                def gbody(i, _):
                    idx = feat_v[pl.ds(p * PIECE + i * 16, 16)]
                    buf[pl.ds(i * 16, 16)] = plsc.load_gather(vvec, [idx])
                    return 0

                lax.fori_loop(0, PIECE // 16, gbody, 0, unroll=4)
                wd[p % 2] = pltpu.async_copy(
                    buf, out_hbm.at[row, pl.ds(p * PIECE, PIECE)], wsem)
            for d in wd:
                if d is not None:
                    d.wait()

    return k(features_t, tables_em)


def _mlp_t(xts, W1s, b1, W2, b2):
    BT = 2048
    grid = (BATCH // BT,)
    n = len(xts)

    def body(*refs):
        xt_refs = refs[:n]
        w1_refs = refs[n:2 * n]
        b1_ref, w2_ref, b2_ref, out_ref = refs[2 * n:]
        h = b1_ref[...]
        for xr, wr in zip(xt_refs, w1_refs):
            h = h + lax.dot_general(wr[...], xr[...],
                                    (((0,), (0,)), ((), ())),
                                    preferred_element_type=jnp.float32)
        h = jnp.maximum(h, 0.0)
        y = lax.dot_general(h, w2_ref[...],
                            (((0,), (0,)), ((), ())),
                            preferred_element_type=jnp.float32)
        out_ref[...] = y + b2_ref[...]

    in_specs = (
        [pl.BlockSpec((xt.shape[0], BT), lambda i: (0, i)) for xt in xts]
        + [pl.BlockSpec(w.shape, lambda i: (0, 0)) for w in W1s]
        + [
            pl.BlockSpec((HIDDEN, 1), lambda i: (0, 0)),
            pl.BlockSpec((HIDDEN, OUT), lambda i: (0, 0)),
            pl.BlockSpec((1, OUT), lambda i: (0, 0)),
        ]
    )
    return pl.pallas_call(
        body,
        grid=grid,
        in_specs=in_specs,
        out_specs=pl.BlockSpec((BT, OUT), lambda i: (i, 0)),
        out_shape=jax.ShapeDtypeStruct((BATCH, OUT), jnp.float32),
    )(*xts, *W1s, b1.reshape(HIDDEN, 1), W2, b2.reshape(1, OUT))


def kernel(features, tables, W1, b1, W2, b2):
    features_t = features.T.astype(jnp.int32)
    tables_em = jnp.transpose(tables, (0, 2, 1))
    xts, W1s = [], []
    f0 = 0
    for fg in GROUPS:
        xts.append(_sc_gather_t(features_t[f0:f0 + fg],
                                tables_em[f0:f0 + fg], fg))
        W1s.append(W1[f0 * EMBED_DIM:(f0 + fg) * EMBED_DIM])
        f0 += fg
    return _mlp_t(xts, W1s, b1, W2, b2)


# speedup vs baseline: 2.0325x; 2.0325x over previous
import functools

import jax
import jax.numpy as jnp
from jax import lax
from jax.experimental import pallas as pl
from jax.experimental.pallas import tpu as pltpu
from jax.experimental.pallas import tpu_sc as plsc

N_FIELDS = 26
VOCAB = 100000
EMBED_DIM = 32
BATCH = 16384
HIDDEN = 512
OUT = 128
INPUT_DIM = N_FIELDS * EMBED_DIM

NC = 2
NS = 16
NW = NC * NS

PIECE = 4096
NPIECE = BATCH // PIECE


def _sc_gather_t(features_t, tables_em):
    mesh = plsc.VectorSubcoreMesh(core_axis_name="c", subcore_axis_name="s",
                                  num_cores=NC, num_subcores=NS)

    @functools.partial(
        pl.kernel,
        out_type=jax.ShapeDtypeStruct((INPUT_DIM, BATCH), jnp.float32),
        mesh=mesh,
        scratch_types=[
            pltpu.VMEM((VOCAB,), jnp.float32),
            pltpu.VMEM((BATCH,), jnp.int32),
            [pltpu.VMEM((PIECE,), jnp.float32) for _ in range(2)],
            pltpu.SemaphoreType.DMA,
        ],
        compiler_params=pltpu.CompilerParams(use_tc_tiling_on_sc=True,
                                             needs_layout_passes=False),
    )
    def k(feat_hbm, tab_hbm, out_hbm, vvec, feat_v, piece_v, wsem):
        e = lax.axis_index("s") * NC + lax.axis_index("c")

        for f in range(N_FIELDS):
            pltpu.sync_copy(tab_hbm.at[f, e], vvec)
            pltpu.sync_copy(feat_hbm.at[f], feat_v)
            row = f * EMBED_DIM + e
            wd = [None, None]
            for p in range(NPIECE):
                buf = piece_v[p % 2]
                if wd[p % 2] is not None:
                    wd[p % 2].wait()

                def gbody(i, _):
                    idx = feat_v[pl.ds(p * PIECE + i * 16, 16)]
                    buf[pl.ds(i * 16, 16)] = plsc.load_gather(vvec, [idx])
                    return 0

                lax.fori_loop(0, PIECE // 16, gbody, 0, unroll=4)
                wd[p % 2] = pltpu.async_copy(
                    buf, out_hbm.at[row, pl.ds(p * PIECE, PIECE)], wsem)
            for d in wd:
                if d is not None:
                    d.wait()

    return k(features_t, tables_em)


def _mlp_t(xt, W1, b1, W2, b2):
    BT = 2048
    grid = (BATCH // BT,)

    def body(xt_ref, w1_ref, b1_ref, w2_ref, b2_ref, out_ref):
        h = lax.dot_general(w1_ref[...], xt_ref[...],
                            (((0,), (0,)), ((), ())),
                            preferred_element_type=jnp.float32)
        h = jnp.maximum(h + b1_ref[...], 0.0)
        y = lax.dot_general(h, w2_ref[...],
                            (((0,), (0,)), ((), ())),
                            preferred_element_type=jnp.float32)
        out_ref[...] = y + b2_ref[...]

    return pl.pallas_call(
        body,
        grid=grid,
        in_specs=[
            pl.BlockSpec((INPUT_DIM, BT), lambda i: (0, i)),
            pl.BlockSpec((INPUT_DIM, HIDDEN), lambda i: (0, 0)),
            pl.BlockSpec((HIDDEN, 1), lambda i: (0, 0)),
            pl.BlockSpec((HIDDEN, OUT), lambda i: (0, 0)),
            pl.BlockSpec((1, OUT), lambda i: (0, 0)),
        ],
        out_specs=pl.BlockSpec((BT, OUT), lambda i: (i, 0)),
        out_shape=jax.ShapeDtypeStruct((BATCH, OUT), jnp.float32),
    )(xt, W1, b1.reshape(HIDDEN, 1), W2, b2.reshape(1, OUT))


def kernel(features, tables, W1, b1, W2, b2):
    features_t = features.T.astype(jnp.int32)
    tables_em = jnp.transpose(tables, (0, 2, 1))
    xt = _sc_gather_t(features_t, tables_em)
    return _mlp_t(xt, W1, b1, W2, b2)


# speedup vs baseline: 2.1553x; 1.0604x over previous
import functools

import jax
import jax.numpy as jnp
from jax import lax
from jax.experimental import pallas as pl
from jax.experimental.pallas import tpu as pltpu
from jax.experimental.pallas import tpu_sc as plsc

N_FIELDS = 26
VOCAB = 100000
EMBED_DIM = 32
BATCH = 16384
HIDDEN = 512
OUT = 128
INPUT_DIM = N_FIELDS * EMBED_DIM

NC = 2
NS = 16
NW = NC * NS

PIECE = 4096
NPIECE = BATCH // PIECE


def _sc_gather_t(features_t, tables_em):
    mesh = plsc.VectorSubcoreMesh(core_axis_name="c", subcore_axis_name="s",
                                  num_cores=NC, num_subcores=NS)

    @functools.partial(
        pl.kernel,
        out_type=jax.ShapeDtypeStruct((INPUT_DIM, BATCH), jnp.float32),
        mesh=mesh,
        scratch_types=[
            pltpu.VMEM((VOCAB,), jnp.float32),
            pltpu.VMEM((BATCH,), jnp.int32),
            [pltpu.VMEM((PIECE,), jnp.float32) for _ in range(2)],
            pltpu.SemaphoreType.DMA,
            pltpu.SemaphoreType.DMA,
        ],
        compiler_params=pltpu.CompilerParams(use_tc_tiling_on_sc=True,
                                             needs_layout_passes=False),
    )
    def k(feat_hbm, tab_hbm, out_hbm, vvec, feat_v, piece_v, gsem, wsem):
        e = lax.axis_index("s") * NC + lax.axis_index("c")
        VH = VOCAB // 2

        def fetch(f):
            return [
                pltpu.async_copy(tab_hbm.at[f, e], vvec, gsem),
                pltpu.async_copy(feat_hbm.at[f], feat_v, gsem),
            ]

        gd = fetch(0)
        wd = [None, None]
        for f in range(N_FIELDS):
            for d in gd:
                d.wait()
            row = f * EMBED_DIM + e
            for p in range(NPIECE):
                buf = piece_v[p % 2]
                if wd[p % 2] is not None:
                    wd[p % 2].wait()

                def gbody(i, _):
                    idx = feat_v[pl.ds(p * PIECE + i * 16, 16)]
                    buf[pl.ds(i * 16, 16)] = plsc.load_gather(vvec, [idx])
                    return 0

                lax.fori_loop(0, PIECE // 16, gbody, 0, unroll=8)
                wd[p % 2] = pltpu.async_copy(
                    buf, out_hbm.at[row, pl.ds(p * PIECE, PIECE)], wsem)
            if f + 1 < N_FIELDS:
                gd = fetch(f + 1)
        for d in wd:
            if d is not None:
                d.wait()

    return k(features_t, tables_em)


def _mlp_t(xt, W1, b1, W2, b2):
    BT = 2048
    grid = (BATCH // BT,)

    def body(xt_ref, w1_ref, b1_ref, w2_ref, b2_ref, out_ref):
        h = lax.dot_general(w1_ref[...], xt_ref[...],
                            (((0,), (0,)), ((), ())),
                            preferred_element_type=jnp.float32)
        h = jnp.maximum(h + b1_ref[...], 0.0)
        y = lax.dot_general(h, w2_ref[...],
                            (((0,), (0,)), ((), ())),
                            preferred_element_type=jnp.float32)
        out_ref[...] = y + b2_ref[...]

    return pl.pallas_call(
        body,
        grid=grid,
        in_specs=[
            pl.BlockSpec((INPUT_DIM, BT), lambda i: (0, i)),
            pl.BlockSpec((INPUT_DIM, HIDDEN), lambda i: (0, 0)),
            pl.BlockSpec((HIDDEN, 1), lambda i: (0, 0)),
            pl.BlockSpec((HIDDEN, OUT), lambda i: (0, 0)),
            pl.BlockSpec((1, OUT), lambda i: (0, 0)),
        ],
        out_specs=pl.BlockSpec((BT, OUT), lambda i: (i, 0)),
        out_shape=jax.ShapeDtypeStruct((BATCH, OUT), jnp.float32),
    )(xt, W1, b1.reshape(HIDDEN, 1), W2, b2.reshape(1, OUT))


def kernel(features, tables, W1, b1, W2, b2):
    features_t = features.T.astype(jnp.int32)
    tables_em = jnp.transpose(tables, (0, 2, 1))
    xt = _sc_gather_t(features_t, tables_em)
    return _mlp_t(xt, W1, b1, W2, b2)


# speedup vs baseline: 3.7601x; 1.7446x over previous
import functools

import jax
import jax.numpy as jnp
from jax import lax
from jax.experimental import pallas as pl
from jax.experimental.pallas import tpu as pltpu
from jax.experimental.pallas import tpu_sc as plsc

N_FIELDS = 26
VOCAB = 100000
EMBED_DIM = 32
BATCH = 16384
HIDDEN = 512
OUT = 128
INPUT_DIM = N_FIELDS * EMBED_DIM

NC = 2
NS = 16
NW = NC * NS

PIECE = 4096
NPIECE = BATCH // PIECE


def _sc_gather_t(features_t, tables_em):
    mesh = plsc.VectorSubcoreMesh(core_axis_name="c", subcore_axis_name="s",
                                  num_cores=NC, num_subcores=NS)

    @functools.partial(
        pl.kernel,
        out_type=jax.ShapeDtypeStruct((INPUT_DIM, BATCH), jnp.float32),
        mesh=mesh,
        scratch_types=[
            pltpu.VMEM((VOCAB,), jnp.float32),
            pltpu.VMEM((BATCH,), jnp.int32),
            [pltpu.VMEM((PIECE,), jnp.float32) for _ in range(2)],
            pltpu.SemaphoreType.DMA,
            pltpu.SemaphoreType.DMA,
        ],
        compiler_params=pltpu.CompilerParams(use_tc_tiling_on_sc=True,
                                             needs_layout_passes=False),
    )
    def k(feat_hbm, tab_hbm, out_hbm, vvec, feat_v, piece_v, gsem, wsem):
        e = lax.axis_index("s") * NC + lax.axis_index("c")
        VH = VOCAB // 2

        def fetch(f):
            return [
                pltpu.async_copy(tab_hbm.at[f, e], vvec, gsem),
                pltpu.async_copy(feat_hbm.at[f], feat_v, gsem),
            ]

        gd = fetch(0)
        wd = [None, None]
        for f in range(N_FIELDS):
            for d in gd:
                d.wait()
            row = f * EMBED_DIM + e
            for p in range(NPIECE):
                buf = piece_v[p % 2]
                if wd[p % 2] is not None:
                    wd[p % 2].wait()
                feat_p = feat_v.at[pl.ds(p * PIECE, PIECE)]

                @plsc.parallel_loop(0, PIECE // 16, 1, unroll=8)
                def gbody(i):
                    idx = feat_p[pl.ds(i * 16, 16)]
                    buf[pl.ds(i * 16, 16)] = plsc.load_gather(vvec, [idx])
                wd[p % 2] = pltpu.async_copy(
                    buf, out_hbm.at[row, pl.ds(p * PIECE, PIECE)], wsem)
            if f + 1 < N_FIELDS:
                gd = fetch(f + 1)
        for d in wd:
            if d is not None:
                d.wait()

    return k(features_t, tables_em)


def _mlp_t(xt, W1, b1, W2, b2):
    BT = 2048
    grid = (BATCH // BT,)

    def body(xt_ref, w1_ref, b1_ref, w2_ref, b2_ref, out_ref):
        h = lax.dot_general(w1_ref[...], xt_ref[...],
                            (((0,), (0,)), ((), ())),
                            preferred_element_type=jnp.float32)
        h = jnp.maximum(h + b1_ref[...], 0.0)
        y = lax.dot_general(h, w2_ref[...],
                            (((0,), (0,)), ((), ())),
                            preferred_element_type=jnp.float32)
        out_ref[...] = y + b2_ref[...]

    return pl.pallas_call(
        body,
        grid=grid,
        in_specs=[
            pl.BlockSpec((INPUT_DIM, BT), lambda i: (0, i)),
            pl.BlockSpec((INPUT_DIM, HIDDEN), lambda i: (0, 0)),
            pl.BlockSpec((HIDDEN, 1), lambda i: (0, 0)),
            pl.BlockSpec((HIDDEN, OUT), lambda i: (0, 0)),
            pl.BlockSpec((1, OUT), lambda i: (0, 0)),
        ],
        out_specs=pl.BlockSpec((BT, OUT), lambda i: (i, 0)),
        out_shape=jax.ShapeDtypeStruct((BATCH, OUT), jnp.float32),
    )(xt, W1, b1.reshape(HIDDEN, 1), W2, b2.reshape(1, OUT))


def kernel(features, tables, W1, b1, W2, b2):
    features_t = features.T.astype(jnp.int32)
    tables_em = jnp.transpose(tables, (0, 2, 1))
    xt = _sc_gather_t(features_t, tables_em)
    return _mlp_t(xt, W1, b1, W2, b2)


# speedup vs baseline: 3.7615x; 1.0004x over previous
import functools

import jax
import jax.numpy as jnp
from jax import lax
from jax.experimental import pallas as pl
from jax.experimental.pallas import tpu as pltpu
from jax.experimental.pallas import tpu_sc as plsc

N_FIELDS = 26
VOCAB = 100000
EMBED_DIM = 32
BATCH = 16384
HIDDEN = 512
OUT = 128
INPUT_DIM = N_FIELDS * EMBED_DIM

NC = 2
NS = 16
NW = NC * NS

PIECE = 4096
NPIECE = BATCH // PIECE


def _sc_gather_t(features_t, tables_em):
    mesh = plsc.VectorSubcoreMesh(core_axis_name="c", subcore_axis_name="s",
                                  num_cores=NC, num_subcores=NS)

    @functools.partial(
        pl.kernel,
        out_type=jax.ShapeDtypeStruct((INPUT_DIM, BATCH), jnp.float32),
        mesh=mesh,
        scratch_types=[
            pltpu.VMEM((VOCAB,), jnp.float32),
            pltpu.VMEM((BATCH,), jnp.int32),
            [pltpu.VMEM((PIECE,), jnp.float32) for _ in range(2)],
            pltpu.SemaphoreType.DMA,
            pltpu.SemaphoreType.DMA,
        ],
        compiler_params=pltpu.CompilerParams(use_tc_tiling_on_sc=True,
                                             needs_layout_passes=False),
    )
    def k(feat_hbm, tab_hbm, out_hbm, vvec, feat_v, piece_v, gsem, wsem):
        e = lax.axis_index("s") * NC + lax.axis_index("c")

        def fetch(f):
            return [
                pltpu.async_copy(tab_hbm.at[f, e], vvec, gsem),
                pltpu.async_copy(feat_hbm.at[f], feat_v, gsem),
            ]

        gd = fetch(0)
        wd = [None, None]
        for f in range(N_FIELDS):
            for d in gd:
                d.wait()
            row = f * EMBED_DIM + e
            for p in range(NPIECE):
                buf = piece_v[p % 2]
                if wd[p % 2] is not None:
                    wd[p % 2].wait()
                feat_p = feat_v.at[pl.ds(p * PIECE, PIECE)]

                @plsc.parallel_loop(0, PIECE // 16, 1, unroll=8)
                def gbody(i):
                    idx = feat_p[pl.ds(i * 16, 16)]
                    buf[pl.ds(i * 16, 16)] = plsc.load_gather(vvec, [idx])
                wd[p % 2] = pltpu.async_copy(
                    buf, out_hbm.at[row, pl.ds(p * PIECE, PIECE)], wsem)
            if f + 1 < N_FIELDS:
                gd = fetch(f + 1)
        for d in wd:
            if d is not None:
                d.wait()

    return k(features_t, tables_em)


def _mlp_t(xt, W1, b1, W2, b2):
    BT = 2048
    grid = (BATCH // BT,)

    def body(xt_ref, w1_ref, b1_ref, w2_ref, b2_ref, out_ref):
        h = lax.dot_general(w1_ref[...], xt_ref[...],
                            (((0,), (0,)), ((), ())),
                            preferred_element_type=jnp.float32)
        h = jnp.maximum(h + b1_ref[...], 0.0)
        y = lax.dot_general(h, w2_ref[...],
                            (((0,), (0,)), ((), ())),
                            preferred_element_type=jnp.float32)
        out_ref[...] = y + b2_ref[...]

    return pl.pallas_call(
        body,
        grid=grid,
        in_specs=[
            pl.BlockSpec((INPUT_DIM, BT), lambda i: (0, i)),
            pl.BlockSpec((INPUT_DIM, HIDDEN), lambda i: (0, 0)),
            pl.BlockSpec((HIDDEN, 1), lambda i: (0, 0)),
            pl.BlockSpec((HIDDEN, OUT), lambda i: (0, 0)),
            pl.BlockSpec((1, OUT), lambda i: (0, 0)),
        ],
        out_specs=pl.BlockSpec((BT, OUT), lambda i: (i, 0)),
        out_shape=jax.ShapeDtypeStruct((BATCH, OUT), jnp.float32),
    )(xt, W1, b1.reshape(HIDDEN, 1), W2, b2.reshape(1, OUT))


def kernel(features, tables, W1, b1, W2, b2):
    features_t = features.T.astype(jnp.int32)
    tables_em = jnp.transpose(tables, (0, 2, 1))
    xt = _sc_gather_t(features_t, tables_em)
    return _mlp_t(xt, W1, b1, W2, b2)
